# baseline (speedup 1.0000x reference)
import functools

import jax
import jax.numpy as jnp
from jax.experimental import pallas as pl
from jax.experimental.pallas import tpu as pltpu


def _body(e_ref, eb_ref, w1_ref, w2_ref, b1_ref, b2_ref, a_ref,
          out_ref, h_ref, *, bi, nm1, de):
    g = pl.program_id(0)
    c = pl.program_id(1)
    hid = h_ref.shape[0] - 1

    @pl.when((c == 0) & (g == 0))
    def _ones_row():
        h_ref[hid:hid + 1, :] = jnp.ones((1, bi * nm1), jnp.bfloat16)

    @pl.when(c == 0)
    def _build_hidden():
        w1 = w1_ref[...]
        et = jnp.transpose(e_ref[...])
        zj0t = jnp.dot(w1[:, de:], et[:, :nm1],
                       preferred_element_type=jnp.float32)
        zj1t = jnp.dot(w1[:, de:], et[:, 1:],
                       preferred_element_type=jnp.float32)
        i0 = g * bi
        eb = jnp.transpose(eb_ref[...])
        zit = (jnp.dot(w1[:, :de], eb,
                       preferred_element_type=jnp.float32)
               + b1_ref[...]).astype(jnp.bfloat16)
        zj0b = zj0t.astype(jnp.bfloat16)
        zj1b = zj1t.astype(jnp.bfloat16)
        a = a_ref[...]
        m0 = (jnp.abs(a[:, :nm1]) > 0).astype(jnp.bfloat16)
        m1 = (jnp.abs(a[:, 1:]) > 0).astype(jnp.bfloat16)
        t_1 = jax.lax.broadcasted_iota(jnp.int32, (1, nm1), 1)
        zero = jnp.zeros((), jnp.bfloat16)
        for u in range(bi):
            i_s = i0 + u
            cond = t_1 < i_s
            zjc = jnp.where(cond, zj0b, zj1b)
            m = jnp.where(cond, m0[u:u + 1, :], m1[u:u + 1, :])
            pre = m * (zit[:, u:u + 1] + zjc)
            h_ref[:hid, u * nm1:(u + 1) * nm1] = jnp.maximum(pre, zero)

    w2full = jnp.concatenate([w2_ref[...], b2_ref[...]], axis=1)
    out_ref[...] = jnp.dot(w2full.astype(jnp.bfloat16), h_ref[...],
                           preferred_element_type=jnp.float32)


def kernel(edge_features, adjacency_matrix, W1, b1, W2, b2):
    n, de = edge_features.shape
    hidden = W1.shape[0]
    dd = W2.shape[0]
    nm1 = n - 1
    bi = 128
    gi = n // bi
    cs = 128
    gc = dd // cs

    b1c = b1.reshape(hidden, 1)
    b2c = b2.reshape(dd, 1)

    out = pl.pallas_call(
        functools.partial(_body, bi=bi, nm1=nm1, de=de),
        grid=(gi, gc),
        in_specs=[
            pl.BlockSpec((n, de), lambda g, c: (0, 0)),
            pl.BlockSpec((bi, de), lambda g, c: (g, 0)),
            pl.BlockSpec((hidden, 2 * de), lambda g, c: (0, 0)),
            pl.BlockSpec((cs, hidden), lambda g, c: (c, 0)),
            pl.BlockSpec((hidden, 1), lambda g, c: (0, 0)),
            pl.BlockSpec((cs, 1), lambda g, c: (c, 0)),
            pl.BlockSpec((bi, n), lambda g, c: (g, 0)),
        ],
        out_specs=pl.BlockSpec((cs, bi * nm1), lambda g, c: (c, g)),
        out_shape=jax.ShapeDtypeStruct((dd, n * nm1), jnp.float32),
        scratch_shapes=[pltpu.VMEM((hidden + 1, bi * nm1), jnp.bfloat16)],
    )(edge_features, edge_features, W1, W2, b1c, b2c, adjacency_matrix)

    sd = int(round(dd ** 0.5))
    return out.reshape(sd, sd, n * nm1).transpose(2, 0, 1)


# speedup vs baseline: 1.0231x; 1.0231x over previous
import functools

import jax
import jax.numpy as jnp
from jax.experimental import pallas as pl
from jax.experimental.pallas import tpu as pltpu


def _body(et_ref, ebt_ref, w1_ref, w2b_ref, b1_ref, a_ref,
          out_ref, h_ref, *, bi, nm1, de):
    g = pl.program_id(0)
    c = pl.program_id(1)
    hid = h_ref.shape[0] - 1

    @pl.when((c == 0) & (g == 0))
    def _ones_row():
        h_ref[hid:hid + 1, :] = jnp.ones((1, bi * nm1), jnp.bfloat16)

    @pl.when(c == 0)
    def _build_hidden():
        w1 = w1_ref[...]
        et = et_ref[...]
        zj0t = jnp.dot(w1[:, de:], et[:, :nm1],
                       preferred_element_type=jnp.float32)
        zj1t = jnp.dot(w1[:, de:], et[:, 1:],
                       preferred_element_type=jnp.float32)
        i0 = g * bi
        zit = (jnp.dot(w1[:, :de], ebt_ref[...],
                       preferred_element_type=jnp.float32)
               + b1_ref[...]).astype(jnp.bfloat16)
        zj0b = zj0t.astype(jnp.bfloat16)
        zj1b = zj1t.astype(jnp.bfloat16)
        a = a_ref[...]
        m0 = (jnp.abs(a[:, :nm1]) > 0).astype(jnp.bfloat16)
        m1 = (jnp.abs(a[:, 1:]) > 0).astype(jnp.bfloat16)
        t_1 = jax.lax.broadcasted_iota(jnp.int32, (1, nm1), 1)
        zero = jnp.zeros((), jnp.bfloat16)
        for u in range(bi):
            i_s = i0 + u
            cond = t_1 < i_s
            zjc = jnp.where(cond, zj0b, zj1b)
            m = jnp.where(cond, m0[u:u + 1, :], m1[u:u + 1, :])
            pre = m * (zit[:, u:u + 1] + zjc)
            h_ref[:hid, u * nm1:(u + 1) * nm1] = jnp.maximum(pre, zero)

    out_ref[...] = jnp.dot(w2b_ref[...].astype(jnp.bfloat16), h_ref[...],
                           preferred_element_type=jnp.float32)


def kernel(edge_features, adjacency_matrix, W1, b1, W2, b2):
    n, de = edge_features.shape
    hidden = W1.shape[0]
    dd = W2.shape[0]
    nm1 = n - 1
    bi = 128
    gi = n // bi
    cs = 128
    gc = dd // cs

    et = edge_features.T
    b1c = b1.reshape(hidden, 1)
    w2b = jnp.concatenate([W2, b2.reshape(dd, 1)], axis=1)

    out = pl.pallas_call(
        functools.partial(_body, bi=bi, nm1=nm1, de=de),
        grid=(gi, gc),
        in_specs=[
            pl.BlockSpec((de, n), lambda g, c: (0, 0)),
            pl.BlockSpec((de, bi), lambda g, c: (0, g)),
            pl.BlockSpec((hidden, 2 * de), lambda g, c: (0, 0)),
            pl.BlockSpec((cs, hidden + 1), lambda g, c: (c, 0)),
            pl.BlockSpec((hidden, 1), lambda g, c: (0, 0)),
            pl.BlockSpec((bi, n), lambda g, c: (g, 0)),
        ],
        out_specs=pl.BlockSpec((cs, bi * nm1), lambda g, c: (c, g)),
        out_shape=jax.ShapeDtypeStruct((dd, n * nm1), jnp.float32),
        scratch_shapes=[pltpu.VMEM((hidden + 1, bi * nm1), jnp.bfloat16)],
    )(et, et, W1, w2b, b1c, adjacency_matrix)

    sd = int(round(dd ** 0.5))
    return out.reshape(sd, sd, n * nm1).transpose(2, 0, 1)
